# baseline (speedup 1.0000x reference)
import ml_dtypes
import numpy as np
import jax
import jax.numpy as jnp
from jax.experimental import pallas as pl
from jax.experimental.pallas import tpu as pltpu

H = 512
W = 512
H2 = 2 * H


def _dfb_taps():
    v = np.array([0.63, -0.193, 0.0972, -0.0526, 0.0272, -0.0144],
                 dtype=np.float32)
    f = np.concatenate((v[::-1], v))
    f[::2] = -f[::2]
    return [float(t) for t in f]


_TAPS = _dfb_taps()
_SQRT2 = float(np.sqrt(2.0))


def _conv_mats(off):
    cv = np.zeros((H, H), dtype=np.float32)
    r = np.arange(H)
    for t in range(12):
        cv[r, (r + off + t) % H] = _TAPS[t]
    ch = np.zeros((W, W), dtype=np.float32)
    for t in range(12):
        ch[(r + off + t) % W, r] = _TAPS[t]
    return cv, ch


_CV0, _CH0 = _conv_mats(-5)
_CV1, _CH1 = _conv_mats(-6)


def _interleave_mat():
    pm = np.zeros((H2, H2), dtype=np.float32)
    q = np.arange(H)
    pm[2 * q, q] = 1.0
    pm[2 * q + 1, H + q] = 1.0
    return pm


_PM = _interleave_mat()

_MATS = tuple(m.astype(ml_dtypes.bfloat16)
              for m in (_CV0, _CH0, _CV1, _CH1, _PM))


def _circ_sep_conv(x, cv, ch):
    xh = x.astype(jnp.bfloat16)
    y = jax.lax.dot(cv, xh, preferred_element_type=jnp.float32)
    return jax.lax.dot(y.astype(jnp.bfloat16), ch,
                       preferred_element_type=jnp.float32)


_CPB = 1


def _body(y0_ref, y1_ref, cv0_ref, ch0_ref, cv1_ref, ch1_ref,
          pm_ref, out_ref):
    ic = jax.lax.broadcasted_iota(jnp.int32, (H2, W), 1)
    for k in range(_CPB):
        x0 = y0_ref[k]
        a = _circ_sep_conv(x0, cv0_ref[...], ch0_ref[...])
        p1 = (-1.0 / _SQRT2) * (y1_ref[k] + a)
        b = _circ_sep_conv(p1, cv1_ref[...], ch1_ref[...])
        p0 = _SQRT2 * x0 + b

        s1 = pltpu.roll(p0, 0, 1, stride=1, stride_axis=0)
        s2 = pltpu.roll(p1, 1, 1, stride=1, stride_axis=0)

        s = jnp.concatenate([s1, s2], axis=0).astype(jnp.bfloat16)
        y = jax.lax.dot(pm_ref[...], s, preferred_element_type=jnp.float32)

        for bit in range(9):
            rolled = pltpu.roll(y, H2 - (1 << bit), 0)
            y = jnp.where((ic << (31 - bit)) < 0, rolled, y)
        out_ref[k] = y


@jax.jit
def kernel(y0, y1):
    n, c = y0.shape[0], y0.shape[1]
    a = y0.reshape(n * c, H, W)
    b = y1.reshape(n * c, H, W)
    out = pl.pallas_call(
        _body,
        grid=(n * c // _CPB,),
        in_specs=[
            pl.BlockSpec((_CPB, H, W), lambda i: (i, 0, 0)),
            pl.BlockSpec((_CPB, H, W), lambda i: (i, 0, 0)),
        ] + [pl.BlockSpec((H, W), lambda i: (0, 0))] * 4
          + [pl.BlockSpec((H2, H2), lambda i: (0, 0))],
        out_specs=pl.BlockSpec((_CPB, H2, W), lambda i: (i, 0, 0)),
        out_shape=jax.ShapeDtypeStruct((n * c, H2, W), jnp.float32),
    )(a, b, *[jnp.asarray(m) for m in _MATS])
    return out.reshape(n, c, H2, W)


# speedup vs baseline: 1.0385x; 1.0385x over previous
import ml_dtypes
import numpy as np
import jax
import jax.numpy as jnp
from jax.experimental import pallas as pl
from jax.experimental.pallas import tpu as pltpu

H = 512
W = 512
H2 = 2 * H


def _dfb_taps():
    v = np.array([0.63, -0.193, 0.0972, -0.0526, 0.0272, -0.0144],
                 dtype=np.float32)
    f = np.concatenate((v[::-1], v))
    f[::2] = -f[::2]
    return [float(t) for t in f]


_TAPS = _dfb_taps()
_SQRT2 = float(np.sqrt(2.0))


def _conv_mats(off):
    cv = np.zeros((H, H), dtype=np.float32)
    r = np.arange(H)
    for t in range(12):
        cv[r, (r + off + t) % H] = _TAPS[t]
    ch = np.zeros((W, W), dtype=np.float32)
    for t in range(12):
        ch[(r + off + t) % W, r] = _TAPS[t]
    return cv, ch


_CV0, _CH0 = _conv_mats(-5)
_CV1, _CH1 = _conv_mats(-6)


def _interleave_mats():
    p1m = np.zeros((H2, H), dtype=np.float32)
    p2m = np.zeros((H2, H), dtype=np.float32)
    q = np.arange(H)
    p1m[2 * q, q] = 1.0
    p2m[2 * q + 1, q] = 1.0
    return p1m, p2m


_P1M, _P2M = _interleave_mats()

_MATS = tuple(m.astype(ml_dtypes.bfloat16)
              for m in (_CV0, _CH0, _CV1, _CH1, _P1M, _P2M))


def _circ_sep_conv(x, cv, ch):
    xh = x.astype(jnp.bfloat16)
    y = jax.lax.dot(cv, xh, preferred_element_type=jnp.float32)
    return jax.lax.dot(y.astype(jnp.bfloat16), ch,
                       preferred_element_type=jnp.float32)


_CPB = 1


def _body(y0_ref, y1_ref, cv0_ref, ch0_ref, cv1_ref, ch1_ref,
          p1m_ref, p2m_ref, out_ref):
    ic = jax.lax.broadcasted_iota(jnp.int32, (H2, W), 1)
    for k in range(_CPB):
        x0 = y0_ref[k]
        a = _circ_sep_conv(x0, cv0_ref[...], ch0_ref[...])
        p1 = (-1.0 / _SQRT2) * (y1_ref[k] + a)
        b = _circ_sep_conv(p1, cv1_ref[...], ch1_ref[...])
        p0 = _SQRT2 * x0 + b

        s1 = pltpu.roll(p0, 0, 1, stride=1, stride_axis=0)
        s2 = pltpu.roll(p1, 1, 1, stride=1, stride_axis=0)

        y = (jax.lax.dot(p1m_ref[...], s1.astype(jnp.bfloat16),
                         preferred_element_type=jnp.float32) +
             jax.lax.dot(p2m_ref[...], s2.astype(jnp.bfloat16),
                         preferred_element_type=jnp.float32))

        for bit in range(9):
            rolled = pltpu.roll(y, H2 - (1 << bit), 0)
            y = jnp.where((ic << (31 - bit)) < 0, rolled, y)
        out_ref[k] = y


@jax.jit
def kernel(y0, y1):
    n, c = y0.shape[0], y0.shape[1]
    a = y0.reshape(n * c, H, W)
    b = y1.reshape(n * c, H, W)
    out = pl.pallas_call(
        _body,
        grid=(n * c // _CPB,),
        in_specs=[
            pl.BlockSpec((_CPB, H, W), lambda i: (i, 0, 0)),
            pl.BlockSpec((_CPB, H, W), lambda i: (i, 0, 0)),
        ] + [pl.BlockSpec((H, W), lambda i: (0, 0))] * 4
          + [pl.BlockSpec((H2, H), lambda i: (0, 0))] * 2,
        out_specs=pl.BlockSpec((_CPB, H2, W), lambda i: (i, 0, 0)),
        out_shape=jax.ShapeDtypeStruct((n * c, H2, W), jnp.float32),
    )(a, b, *[jnp.asarray(m) for m in _MATS])
    return out.reshape(n, c, H2, W)
